# baseline (speedup 1.0000x reference)
import functools

import jax
import jax.numpy as jnp
from jax import lax
from jax.experimental import pallas as pl
from jax.experimental.pallas import tpu as pltpu
from jax.experimental.pallas import tpu_sc as plsc

B, N, F, D, E, M, L = 8, 2048, 128, 256, 16384, 512, 4
NC, NS = 2, 16
NW = NC * NS
CH = 128

_MESH = plsc.VectorSubcoreMesh(
    core_axis_name="c", subcore_axis_name="s", num_cores=NC, num_subcores=NS)


def _silu(x):
    return x / (1.0 + jnp.exp(-x))


def _embed_body(x_ref, w1_ref, b1_ref, w2_ref, b2_ref, o_ref):
    a = jnp.dot(x_ref[...], w1_ref[...], preferred_element_type=jnp.float32, precision=lax.Precision.HIGHEST)
    a = _silu(a + b1_ref[...])
    o_ref[...] = jnp.dot(a, w2_ref[...],
                         preferred_element_type=jnp.float32, precision=lax.Precision.HIGHEST) + b2_ref[...]


def _tc_embed(x, w1, b1, w2, b2):
    R = x.shape[0]
    BR = 2048
    return pl.pallas_call(
        _embed_body,
        grid=(R // BR,),
        in_specs=[
            pl.BlockSpec((BR, F), lambda i: (i, 0)),
            pl.BlockSpec((F, D), lambda i: (0, 0)),
            pl.BlockSpec((1, D), lambda i: (0, 0)),
            pl.BlockSpec((D, D), lambda i: (0, 0)),
            pl.BlockSpec((1, D), lambda i: (0, 0)),
        ],
        out_specs=pl.BlockSpec((BR, D), lambda i: (i, 0)),
        out_shape=jax.ShapeDtypeStruct((R, D), jnp.float32),
    )(x, w1, b1, w2, b2)


def _pq_body(h_ref, wa_ref, b1_ref, wb_ref, p_ref, q_ref):
    x = h_ref[...]
    p_ref[...] = jnp.dot(x, wa_ref[...],
                         preferred_element_type=jnp.float32) + b1_ref[...]
    q_ref[...] = jnp.dot(x, wb_ref[...], preferred_element_type=jnp.float32)


def _tc_pq(h, wa, b1, wb):
    R = h.shape[0]
    BR = 2048
    return pl.pallas_call(
        _pq_body,
        grid=(R // BR,),
        in_specs=[
            pl.BlockSpec((BR, D), lambda i: (i, 0)),
            pl.BlockSpec((D, D), lambda i: (0, 0)),
            pl.BlockSpec((1, D), lambda i: (0, 0)),
            pl.BlockSpec((D, D), lambda i: (0, 0)),
        ],
        out_specs=[pl.BlockSpec((BR, D), lambda i: (i, 0)),
                   pl.BlockSpec((BR, D), lambda i: (i, 0))],
        out_shape=[jax.ShapeDtypeStruct((R, D), jnp.float32),
                   jax.ShapeDtypeStruct((R, D), jnp.float32)],
    )(h, wa, b1, wb)


EB = 512
EH = E // 4


def _agg_body(t_ref, pt_ref, qs_ref, o_ref):
    s = _silu(pt_ref[...] + qs_ref[...]).astype(jnp.bfloat16)
    t = jnp.clip(t_ref[0], 0, N - 1)
    rows = lax.broadcasted_iota(jnp.int32, (N, EB), 0)
    oh = (rows == t).astype(jnp.bfloat16)
    part = jnp.dot(oh, s, preferred_element_type=jnp.float32)

    @pl.when(pl.program_id(1) == 0)
    def _():
        o_ref[0] = part

    @pl.when(pl.program_id(1) != 0)
    def _():
        o_ref[0] += part


def _tc_agg(tgt3, pt, qs):
    nb = pt.shape[0] // B // EB
    return pl.pallas_call(
        _agg_body,
        grid=(B, nb),
        in_specs=[
            pl.BlockSpec((1, 1, EB), lambda b, e: (b * nb + e, 0, 0)),
            pl.BlockSpec((EB, D), lambda b, e: (b * nb + e, 0)),
            pl.BlockSpec((EB, D), lambda b, e: (b * nb + e, 0)),
        ],
        out_specs=pl.BlockSpec((1, N, D), lambda b, e: (b, 0, 0)),
        out_shape=jax.ShapeDtypeStruct((B, N, D), jnp.float32),
    )(tgt3, pt, qs)


def _deg_body(t_ref, o_ref):
    t = jnp.clip(t_ref[0], 0, N - 1)
    rows = lax.broadcasted_iota(jnp.int32, (N, EB), 0)
    oh = (rows == t).astype(jnp.bfloat16)
    part = jnp.dot(oh, jnp.ones((EB, 128), jnp.bfloat16),
                   preferred_element_type=jnp.float32)

    @pl.when(pl.program_id(1) == 0)
    def _():
        o_ref[0] = part

    @pl.when(pl.program_id(1) != 0)
    def _():
        o_ref[0] += part


def _tc_deg(tgt3):
    nb = E // EB
    return pl.pallas_call(
        _deg_body,
        grid=(B, nb),
        in_specs=[pl.BlockSpec((1, 1, EB), lambda b, e: (b * nb + e, 0, 0))],
        out_specs=pl.BlockSpec((1, N, 128), lambda b, e: (b, 0, 0)),
        out_shape=jax.ShapeDtypeStruct((B, N, 128), jnp.float32),
    )(tgt3)


def _upd_body(h_ref, aggs_ref, aggsb_ref, aggsc_ref, aggsd_ref,
              deg_ref, w2m_ref, b2m_ref,
              wh_ref, wa_ref, b1_ref, w2_ref, b2_ref, g_ref, be_ref, o_ref):
    h = h_ref[...]
    agg = (jnp.dot((aggs_ref[...] + aggsb_ref[...])
                   + (aggsc_ref[...] + aggsd_ref[...]), w2m_ref[...],
                   preferred_element_type=jnp.float32)
           + deg_ref[...][:, 0:1] * b2m_ref[...])
    a = (jnp.dot(h, wh_ref[...], preferred_element_type=jnp.float32)
         + jnp.dot(agg, wa_ref[...],
                   preferred_element_type=jnp.float32) + b1_ref[...])
    a = _silu(a)
    u = jnp.dot(a, w2_ref[...], preferred_element_type=jnp.float32) + b2_ref[...]
    y = h + u
    m = jnp.mean(y, axis=-1, keepdims=True)
    yc = y - m
    v = jnp.mean(yc * yc, axis=-1, keepdims=True)
    o_ref[...] = yc / jnp.sqrt(v + 1e-5) * g_ref[...] + be_ref[...]


def _tc_upd(h, aggs, aggsb, aggsc, aggsd, deg, w2m, b2m,
            wh, wa, b1, w2, b2, g, be):
    R = h.shape[0]
    BR = 2048
    return pl.pallas_call(
        _upd_body,
        grid=(R // BR,),
        in_specs=[
            pl.BlockSpec((BR, D), lambda i: (i, 0)),
            pl.BlockSpec((BR, D), lambda i: (i, 0)),
            pl.BlockSpec((BR, D), lambda i: (i, 0)),
            pl.BlockSpec((BR, D), lambda i: (i, 0)),
            pl.BlockSpec((BR, D), lambda i: (i, 0)),
            pl.BlockSpec((BR, 128), lambda i: (i, 0)),
            pl.BlockSpec((D, D), lambda i: (0, 0)),
            pl.BlockSpec((1, D), lambda i: (0, 0)),
            pl.BlockSpec((D, D), lambda i: (0, 0)),
            pl.BlockSpec((D, D), lambda i: (0, 0)),
            pl.BlockSpec((1, D), lambda i: (0, 0)),
            pl.BlockSpec((D, D), lambda i: (0, 0)),
            pl.BlockSpec((1, D), lambda i: (0, 0)),
            pl.BlockSpec((1, D), lambda i: (0, 0)),
            pl.BlockSpec((1, D), lambda i: (0, 0)),
        ],
        out_specs=pl.BlockSpec((BR, D), lambda i: (i, 0)),
        out_shape=jax.ShapeDtypeStruct((R, D), jnp.float32),
    )(h, aggs, aggsb, aggsc, aggsd, deg, w2m, b2m, wh, wa, b1, w2, b2, g, be)


def _scorer_body(x_ref, w1_ref, b1_ref, w2_ref, b2_ref, w3_ref, b3_ref,
                 mask_ref, o_ref):
    a = _silu(jnp.dot(x_ref[...], w1_ref[...],
                      preferred_element_type=jnp.float32, precision=lax.Precision.HIGHEST) + b1_ref[...])
    a = _silu(jnp.dot(a, w2_ref[...],
                      preferred_element_type=jnp.float32, precision=lax.Precision.HIGHEST) + b2_ref[...])
    s = jnp.dot(a, w3_ref[...], preferred_element_type=jnp.float32, precision=lax.Precision.HIGHEST) + b3_ref[...]
    o_ref[...] = jnp.where(mask_ref[...] > 0, s, -jnp.inf)


def _tc_scorer(x, w1, b1, w2, b2, w3, b3, maskf):
    R = x.shape[0]
    BR = 512
    return pl.pallas_call(
        _scorer_body,
        grid=(R // BR,),
        in_specs=[
            pl.BlockSpec((BR, 4 * D), lambda i: (i, 0)),
            pl.BlockSpec((4 * D, D), lambda i: (0, 0)),
            pl.BlockSpec((1, D), lambda i: (0, 0)),
            pl.BlockSpec((D, D), lambda i: (0, 0)),
            pl.BlockSpec((1, D), lambda i: (0, 0)),
            pl.BlockSpec((D, 128), lambda i: (0, 0)),
            pl.BlockSpec((1, 128), lambda i: (0, 0)),
            pl.BlockSpec((BR, 128), lambda i: (i, 0)),
        ],
        out_specs=pl.BlockSpec((BR, 128), lambda i: (i, 0)),
        out_shape=jax.ShapeDtypeStruct((R, 128), jnp.float32),
    )(x, w1, b1, w2, b2, w3, b3, maskf)


def _adjust_indices(idx_ref, count, boff):
    def body(i, _):
        v = idx_ref[pl.ds(i * 16, 16)]
        idx_ref[pl.ds(i * 16, 16)] = jnp.clip(v, 0, N - 1) + boff
        return 0
    lax.fori_loop(0, count // 16, body, 0, unroll=4)


def _gather2_body(p_hbm, q_hbm, tgt_hbm, src_hbm, pt_out, qs_out,
                  tidx, sidx, buf0, buf1, sga, sgb, swa, swb):
    per = (B * EH) // NW
    wid = lax.axis_index("s") * NC + lax.axis_index("c")
    base = wid * per
    boff = (base // EH) * N
    pltpu.sync_copy(tgt_hbm.at[pl.ds(base, per)], tidx)
    pltpu.sync_copy(src_hbm.at[pl.ds(base, per)], sidx)
    _adjust_indices(tidx, per, boff)
    _adjust_indices(sidx, per, boff)

    def gath(idx_ref, table, out_hbm):
        def body(i, _):
            c0 = i * 2
            ga = pltpu.async_copy(
                table.at[idx_ref.at[pl.ds(c0 * CH, CH)]], buf0, sga)
            gb = pltpu.async_copy(
                table.at[idx_ref.at[pl.ds((c0 + 1) * CH, CH)]], buf1, sgb)
            ga.wait()
            wa = pltpu.async_copy(
                buf0, out_hbm.at[pl.ds(base + c0 * CH, CH)], swa)
            gb.wait()
            wb = pltpu.async_copy(
                buf1, out_hbm.at[pl.ds(base + (c0 + 1) * CH, CH)], swb)
            wa.wait()
            wb.wait()
            return 0
        lax.fori_loop(0, per // CH // 2, body, 0)

    gath(tidx, p_hbm, pt_out)
    gath(sidx, q_hbm, qs_out)


@functools.partial(
    pl.kernel,
    out_type=[jax.ShapeDtypeStruct((B * EH, D), jnp.float32),
              jax.ShapeDtypeStruct((B * EH, D), jnp.float32)],
    mesh=_MESH,
    scratch_types=[
        pltpu.VMEM(((B * EH) // NW,), jnp.int32),
        pltpu.VMEM(((B * EH) // NW,), jnp.int32),
        pltpu.VMEM((CH, D), jnp.float32),
        pltpu.VMEM((CH, D), jnp.float32),
        pltpu.SemaphoreType.DMA,
        pltpu.SemaphoreType.DMA,
        pltpu.SemaphoreType.DMA,
        pltpu.SemaphoreType.DMA,
    ],
)
def _sc_gather2(p_hbm, q_hbm, tgt_hbm, src_hbm, pt_out, qs_out,
                tidx, sidx, buf0, buf1, sga, sgb, swa, swb):
    _gather2_body(p_hbm, q_hbm, tgt_hbm, src_hbm, pt_out, qs_out,
                  tidx, sidx, buf0, buf1, sga, sgb, swa, swb)


@functools.partial(
    pl.kernel,
    out_type=jax.ShapeDtypeStruct((B * M * 4, D), jnp.float32),
    mesh=_MESH,
    scratch_types=[
        pltpu.VMEM(((B * M * 4) // NW,), jnp.int32),
        pltpu.VMEM((CH, D), jnp.float32),
        pltpu.SemaphoreType.DMA,
    ],
)
def _sc_gather_moves(h_hbm, mv_hbm, out_hbm, midx, buf, sem):
    per = (B * M * 4) // NW
    wid = lax.axis_index("s") * NC + lax.axis_index("c")
    base = wid * per
    boff = (base // (M * 4)) * N
    pltpu.sync_copy(mv_hbm.at[pl.ds(base, per)], midx)
    _adjust_indices(midx, per, boff)

    def chunk(c, _):
        cp = pltpu.async_copy(h_hbm.at[midx.at[pl.ds(c * CH, CH)]], buf, sem)
        cp.wait()
        pltpu.sync_copy(buf, out_hbm.at[pl.ds(base + c * CH, CH)])
        return 0
    lax.fori_loop(0, per // CH, chunk, 0)


def kernel(node_features, edge_index, move_nodes, move_mask, params):
    x = node_features.reshape(B * N, F)
    tgt = edge_index[:, :, 1].reshape(B * E).astype(jnp.int32)
    srcs = [edge_index[:, k * EH:(k + 1) * EH, 0].reshape(B * EH).astype(jnp.int32)
            for k in range(4)]
    tgts = [edge_index[:, k * EH:(k + 1) * EH, 1].reshape(B * EH).astype(jnp.int32)
            for k in range(4)]
    mv = move_nodes.reshape(B * M * 4).astype(jnp.int32)

    def w(lin):
        return lin["w"]

    def bvec(lin):
        return lin["b"].reshape(1, -1)

    emb = params["embed"]
    h = _tc_embed(x, w(emb[0]), bvec(emb[0]), w(emb[1]), bvec(emb[1]))

    tgt3 = tgt.reshape(B * E // EB, 1, EB)
    tgt3s = [t.reshape(B * EH // EB, 1, EB) for t in tgts]
    deg = _tc_deg(tgt3).reshape(B * N, 128)
    for lp in params["layers"]:
        m0, m1 = lp["msg"]
        u0, u1 = lp["upd"]
        p, q = _tc_pq(h, m0["w"][:D], bvec(m0), m0["w"][D:])
        gath = [_sc_gather2(p, q, tgts[k], srcs[k]) for k in range(4)]
        parts = [_tc_agg(tgt3s[k], gath[k][0], gath[k][1]).reshape(B * N, D)
                 for k in range(4)]
        h = _tc_upd(h, parts[0], parts[1], parts[2], parts[3],
                    deg, w(m1), bvec(m1),
                    u0["w"][:D], u0["w"][D:], bvec(u0),
                    w(u1), bvec(u1), lp["ln_g"].reshape(1, D),
                    lp["ln_b"].reshape(1, D))

    hm = _sc_gather_moves(h, mv)
    hm4 = hm.reshape(B * M, 4 * D)
    s0, s1, s2 = params["scorer"]
    w3 = jnp.broadcast_to(s2["w"], (D, 128))
    b3 = jnp.broadcast_to(s2["b"].reshape(1, 1), (1, 128))
    maskf = jnp.broadcast_to(
        move_mask.reshape(B * M, 1).astype(jnp.float32), (B * M, 128))
    sc = _tc_scorer(hm4, w(s0), bvec(s0), w(s1), bvec(s1), w3, b3, maskf)
    return sc[:, 0].reshape(B, M)


# speedup vs baseline: 1.0093x; 1.0093x over previous
import functools

import jax
import jax.numpy as jnp
from jax import lax
from jax.experimental import pallas as pl
from jax.experimental.pallas import tpu as pltpu
from jax.experimental.pallas import tpu_sc as plsc

B, N, F, D, E, M, L = 8, 2048, 128, 256, 16384, 512, 4
NC, NS = 2, 16
NW = NC * NS
CH = 128

_MESH = plsc.VectorSubcoreMesh(
    core_axis_name="c", subcore_axis_name="s", num_cores=NC, num_subcores=NS)


def _silu(x):
    return x / (1.0 + jnp.exp(-x))


def _embed_body(x_ref, w1_ref, b1_ref, w2_ref, b2_ref, o_ref):
    a = jnp.dot(x_ref[...], w1_ref[...], preferred_element_type=jnp.float32, precision=lax.Precision.HIGHEST)
    a = _silu(a + b1_ref[...])
    o_ref[...] = jnp.dot(a, w2_ref[...],
                         preferred_element_type=jnp.float32, precision=lax.Precision.HIGHEST) + b2_ref[...]


def _tc_embed(x, w1, b1, w2, b2):
    R = x.shape[0]
    BR = 2048
    return pl.pallas_call(
        _embed_body,
        grid=(R // BR,),
        in_specs=[
            pl.BlockSpec((BR, F), lambda i: (i, 0)),
            pl.BlockSpec((F, D), lambda i: (0, 0)),
            pl.BlockSpec((1, D), lambda i: (0, 0)),
            pl.BlockSpec((D, D), lambda i: (0, 0)),
            pl.BlockSpec((1, D), lambda i: (0, 0)),
        ],
        out_specs=pl.BlockSpec((BR, D), lambda i: (i, 0)),
        out_shape=jax.ShapeDtypeStruct((R, D), jnp.float32),
    )(x, w1, b1, w2, b2)


def _pq_body(h_ref, wa_ref, b1_ref, wb_ref, p_ref, q_ref):
    x = h_ref[...]
    p_ref[...] = jnp.dot(x, wa_ref[...],
                         preferred_element_type=jnp.float32) + b1_ref[...]
    q_ref[...] = jnp.dot(x, wb_ref[...], preferred_element_type=jnp.float32)


def _tc_pq(h, wa, b1, wb):
    R = h.shape[0]
    BR = 2048
    return pl.pallas_call(
        _pq_body,
        grid=(R // BR,),
        in_specs=[
            pl.BlockSpec((BR, D), lambda i: (i, 0)),
            pl.BlockSpec((D, D), lambda i: (0, 0)),
            pl.BlockSpec((1, D), lambda i: (0, 0)),
            pl.BlockSpec((D, D), lambda i: (0, 0)),
        ],
        out_specs=[pl.BlockSpec((BR, D), lambda i: (i, 0)),
                   pl.BlockSpec((BR, D), lambda i: (i, 0))],
        out_shape=[jax.ShapeDtypeStruct((R, D), jnp.float32),
                   jax.ShapeDtypeStruct((R, D), jnp.float32)],
    )(h, wa, b1, wb)


EB = 512
EH = E // 2


def _agg_body(t_ref, pt_ref, qs_ref, o_ref):
    s = _silu(pt_ref[...] + qs_ref[...]).astype(jnp.bfloat16)
    t = jnp.clip(t_ref[0], 0, N - 1)
    rows = lax.broadcasted_iota(jnp.int32, (N, EB), 0)
    oh = (rows == t).astype(jnp.bfloat16)
    part = jnp.dot(oh, s, preferred_element_type=jnp.float32)

    @pl.when(pl.program_id(1) == 0)
    def _():
        o_ref[0] = part

    @pl.when(pl.program_id(1) != 0)
    def _():
        o_ref[0] += part


def _tc_agg(tgt3, pt, qs):
    nb = pt.shape[0] // B // EB
    return pl.pallas_call(
        _agg_body,
        grid=(B, nb),
        in_specs=[
            pl.BlockSpec((1, 1, EB), lambda b, e: (b * nb + e, 0, 0)),
            pl.BlockSpec((EB, D), lambda b, e: (b * nb + e, 0)),
            pl.BlockSpec((EB, D), lambda b, e: (b * nb + e, 0)),
        ],
        out_specs=pl.BlockSpec((1, N, D), lambda b, e: (b, 0, 0)),
        out_shape=jax.ShapeDtypeStruct((B, N, D), jnp.float32),
    )(tgt3, pt, qs)


def _deg_body(t_ref, o_ref):
    t = jnp.clip(t_ref[0], 0, N - 1)
    rows = lax.broadcasted_iota(jnp.int32, (N, EB), 0)
    oh = (rows == t).astype(jnp.bfloat16)
    part = jnp.dot(oh, jnp.ones((EB, 128), jnp.bfloat16),
                   preferred_element_type=jnp.float32)

    @pl.when(pl.program_id(1) == 0)
    def _():
        o_ref[0] = part

    @pl.when(pl.program_id(1) != 0)
    def _():
        o_ref[0] += part


def _tc_deg(tgt3):
    nb = E // EB
    return pl.pallas_call(
        _deg_body,
        grid=(B, nb),
        in_specs=[pl.BlockSpec((1, 1, EB), lambda b, e: (b * nb + e, 0, 0))],
        out_specs=pl.BlockSpec((1, N, 128), lambda b, e: (b, 0, 0)),
        out_shape=jax.ShapeDtypeStruct((B, N, 128), jnp.float32),
    )(tgt3)


def _upd_body(h_ref, aggs_ref, aggsb_ref, deg_ref, w2m_ref, b2m_ref,
              wh_ref, wa_ref, b1_ref, w2_ref, b2_ref, g_ref, be_ref, o_ref):
    h = h_ref[...]
    agg = (jnp.dot(aggs_ref[...] + aggsb_ref[...], w2m_ref[...],
                   preferred_element_type=jnp.float32)
           + deg_ref[...][:, 0:1] * b2m_ref[...])
    a = (jnp.dot(h, wh_ref[...], preferred_element_type=jnp.float32)
         + jnp.dot(agg, wa_ref[...],
                   preferred_element_type=jnp.float32) + b1_ref[...])
    a = _silu(a)
    u = jnp.dot(a, w2_ref[...], preferred_element_type=jnp.float32) + b2_ref[...]
    y = h + u
    m = jnp.mean(y, axis=-1, keepdims=True)
    yc = y - m
    v = jnp.mean(yc * yc, axis=-1, keepdims=True)
    o_ref[...] = yc / jnp.sqrt(v + 1e-5) * g_ref[...] + be_ref[...]


def _tc_upd(h, aggs, aggsb, deg, w2m, b2m, wh, wa, b1, w2, b2, g, be):
    R = h.shape[0]
    BR = 2048
    return pl.pallas_call(
        _upd_body,
        grid=(R // BR,),
        in_specs=[
            pl.BlockSpec((BR, D), lambda i: (i, 0)),
            pl.BlockSpec((BR, D), lambda i: (i, 0)),
            pl.BlockSpec((BR, D), lambda i: (i, 0)),
            pl.BlockSpec((BR, 128), lambda i: (i, 0)),
            pl.BlockSpec((D, D), lambda i: (0, 0)),
            pl.BlockSpec((1, D), lambda i: (0, 0)),
            pl.BlockSpec((D, D), lambda i: (0, 0)),
            pl.BlockSpec((D, D), lambda i: (0, 0)),
            pl.BlockSpec((1, D), lambda i: (0, 0)),
            pl.BlockSpec((D, D), lambda i: (0, 0)),
            pl.BlockSpec((1, D), lambda i: (0, 0)),
            pl.BlockSpec((1, D), lambda i: (0, 0)),
            pl.BlockSpec((1, D), lambda i: (0, 0)),
        ],
        out_specs=pl.BlockSpec((BR, D), lambda i: (i, 0)),
        out_shape=jax.ShapeDtypeStruct((R, D), jnp.float32),
    )(h, aggs, aggsb, deg, w2m, b2m, wh, wa, b1, w2, b2, g, be)


def _scorer_body(x_ref, w1_ref, b1_ref, w2_ref, b2_ref, w3_ref, b3_ref,
                 mask_ref, o_ref):
    a = _silu(jnp.dot(x_ref[...], w1_ref[...],
                      preferred_element_type=jnp.float32, precision=lax.Precision.HIGHEST) + b1_ref[...])
    a = _silu(jnp.dot(a, w2_ref[...],
                      preferred_element_type=jnp.float32, precision=lax.Precision.HIGHEST) + b2_ref[...])
    s = jnp.dot(a, w3_ref[...], preferred_element_type=jnp.float32, precision=lax.Precision.HIGHEST) + b3_ref[...]
    o_ref[...] = jnp.where(mask_ref[...] > 0, s, -jnp.inf)


def _tc_scorer(x, w1, b1, w2, b2, w3, b3, maskf):
    R = x.shape[0]
    BR = 512
    return pl.pallas_call(
        _scorer_body,
        grid=(R // BR,),
        in_specs=[
            pl.BlockSpec((BR, 4 * D), lambda i: (i, 0)),
            pl.BlockSpec((4 * D, D), lambda i: (0, 0)),
            pl.BlockSpec((1, D), lambda i: (0, 0)),
            pl.BlockSpec((D, D), lambda i: (0, 0)),
            pl.BlockSpec((1, D), lambda i: (0, 0)),
            pl.BlockSpec((D, 128), lambda i: (0, 0)),
            pl.BlockSpec((1, 128), lambda i: (0, 0)),
            pl.BlockSpec((BR, 128), lambda i: (i, 0)),
        ],
        out_specs=pl.BlockSpec((BR, 128), lambda i: (i, 0)),
        out_shape=jax.ShapeDtypeStruct((R, 128), jnp.float32),
    )(x, w1, b1, w2, b2, w3, b3, maskf)


def _adjust_indices(idx_ref, count, boff):
    def body(i, _):
        v = idx_ref[pl.ds(i * 16, 16)]
        idx_ref[pl.ds(i * 16, 16)] = jnp.clip(v, 0, N - 1) + boff
        return 0
    lax.fori_loop(0, count // 16, body, 0, unroll=4)


def _gather2_body(p_hbm, q_hbm, tgt_hbm, src_hbm, pt_out, qs_out,
                  tidx, sidx, buf0, buf1, sga, sgb, swa, swb):
    per = (B * EH) // NW
    wid = lax.axis_index("s") * NC + lax.axis_index("c")
    base = wid * per
    boff = (base // EH) * N
    pltpu.sync_copy(tgt_hbm.at[pl.ds(base, per)], tidx)
    pltpu.sync_copy(src_hbm.at[pl.ds(base, per)], sidx)
    _adjust_indices(tidx, per, boff)
    _adjust_indices(sidx, per, boff)

    def gath(idx_ref, table, out_hbm):
        def body(i, _):
            c0 = i * 2
            ga = pltpu.async_copy(
                table.at[idx_ref.at[pl.ds(c0 * CH, CH)]], buf0, sga)
            gb = pltpu.async_copy(
                table.at[idx_ref.at[pl.ds((c0 + 1) * CH, CH)]], buf1, sgb)
            ga.wait()
            wa = pltpu.async_copy(
                buf0, out_hbm.at[pl.ds(base + c0 * CH, CH)], swa)
            gb.wait()
            wb = pltpu.async_copy(
                buf1, out_hbm.at[pl.ds(base + (c0 + 1) * CH, CH)], swb)
            wa.wait()
            wb.wait()
            return 0
        lax.fori_loop(0, per // CH // 2, body, 0)

    gath(tidx, p_hbm, pt_out)
    gath(sidx, q_hbm, qs_out)


@functools.partial(
    pl.kernel,
    out_type=[jax.ShapeDtypeStruct((B * EH, D), jnp.float32),
              jax.ShapeDtypeStruct((B * EH, D), jnp.float32)],
    mesh=_MESH,
    scratch_types=[
        pltpu.VMEM(((B * EH) // NW,), jnp.int32),
        pltpu.VMEM(((B * EH) // NW,), jnp.int32),
        pltpu.VMEM((CH, D), jnp.float32),
        pltpu.VMEM((CH, D), jnp.float32),
        pltpu.SemaphoreType.DMA,
        pltpu.SemaphoreType.DMA,
        pltpu.SemaphoreType.DMA,
        pltpu.SemaphoreType.DMA,
    ],
)
def _sc_gather2(p_hbm, q_hbm, tgt_hbm, src_hbm, pt_out, qs_out,
                tidx, sidx, buf0, buf1, sga, sgb, swa, swb):
    _gather2_body(p_hbm, q_hbm, tgt_hbm, src_hbm, pt_out, qs_out,
                  tidx, sidx, buf0, buf1, sga, sgb, swa, swb)


@functools.partial(
    pl.kernel,
    out_type=jax.ShapeDtypeStruct((B * M * 4, D), jnp.float32),
    mesh=_MESH,
    scratch_types=[
        pltpu.VMEM(((B * M * 4) // NW,), jnp.int32),
        pltpu.VMEM((CH, D), jnp.float32),
        pltpu.SemaphoreType.DMA,
    ],
)
def _sc_gather_moves(h_hbm, mv_hbm, out_hbm, midx, buf, sem):
    per = (B * M * 4) // NW
    wid = lax.axis_index("s") * NC + lax.axis_index("c")
    base = wid * per
    boff = (base // (M * 4)) * N
    pltpu.sync_copy(mv_hbm.at[pl.ds(base, per)], midx)
    _adjust_indices(midx, per, boff)

    def chunk(c, _):
        cp = pltpu.async_copy(h_hbm.at[midx.at[pl.ds(c * CH, CH)]], buf, sem)
        cp.wait()
        pltpu.sync_copy(buf, out_hbm.at[pl.ds(base + c * CH, CH)])
        return 0
    lax.fori_loop(0, per // CH, chunk, 0)


def kernel(node_features, edge_index, move_nodes, move_mask, params):
    x = node_features.reshape(B * N, F)
    tgt = edge_index[:, :, 1].reshape(B * E).astype(jnp.int32)
    srcs = [edge_index[:, k * EH:(k + 1) * EH, 0].reshape(B * EH).astype(jnp.int32)
            for k in range(2)]
    tgts = [edge_index[:, k * EH:(k + 1) * EH, 1].reshape(B * EH).astype(jnp.int32)
            for k in range(2)]
    mv = move_nodes.reshape(B * M * 4).astype(jnp.int32)

    def w(lin):
        return lin["w"]

    def bvec(lin):
        return lin["b"].reshape(1, -1)

    emb = params["embed"]
    h = _tc_embed(x, w(emb[0]), bvec(emb[0]), w(emb[1]), bvec(emb[1]))

    tgt3 = tgt.reshape(B * E // EB, 1, EB)
    tgt3s = [t.reshape(B * EH // EB, 1, EB) for t in tgts]
    deg = _tc_deg(tgt3).reshape(B * N, 128)
    for lp in params["layers"]:
        m0, m1 = lp["msg"]
        u0, u1 = lp["upd"]
        p, q = _tc_pq(h, m0["w"][:D], bvec(m0), m0["w"][D:])
        gath = [_sc_gather2(p, q, tgts[k], srcs[k]) for k in range(2)]
        parts = [_tc_agg(tgt3s[k], gath[k][0], gath[k][1]).reshape(B * N, D)
                 for k in range(2)]
        h = _tc_upd(h, parts[0], parts[1],
                    deg, w(m1), bvec(m1),
                    u0["w"][:D], u0["w"][D:], bvec(u0),
                    w(u1), bvec(u1), lp["ln_g"].reshape(1, D),
                    lp["ln_b"].reshape(1, D))

    hm = _sc_gather_moves(h, mv)
    hm4 = hm.reshape(B * M, 4 * D)
    s0, s1, s2 = params["scorer"]
    w3 = jnp.broadcast_to(s2["w"], (D, 128))
    b3 = jnp.broadcast_to(s2["b"].reshape(1, 1), (1, 128))
    maskf = jnp.broadcast_to(
        move_mask.reshape(B * M, 1).astype(jnp.float32), (B * M, 128))
    sc = _tc_scorer(hm4, w(s0), bvec(s0), w(s1), bvec(s1), w3, b3, maskf)
    return sc[:, 0].reshape(B, M)
